# baseline (speedup 1.0000x reference)
import jax
import jax.numpy as jnp
from jax import lax
from jax.experimental import pallas as pl
from jax.experimental.pallas import tpu as pltpu
from jax.experimental.pallas import tpu_sc as plsc

N_ROWS = 16384
N_COLS = 4096
N_IDX = 2048

NC = 2
NS = 16
NW = NC * NS
L = 16

ROWS_PER_W = N_ROWS // NW
C = 4
NBLK = ROWS_PER_W // C
NSB = NBLK // 2
CHUNKS = N_IDX // L
UNROLL = 8


def _sc_gather_body(x_hbm, d_hbm, out_hbm,
                    x0, x1, d0, d1, o0, o1, si0, si1, so0, so1):
    wid = lax.axis_index("s") * NC + lax.axis_index("c")
    row_base = wid * ROWS_PER_W

    def start_in(blk, xb, db, sem):
        r0 = row_base + blk * C
        pltpu.async_copy(x_hbm.at[pl.ds(r0, C)], xb, sem)
        pltpu.async_copy(d_hbm.at[pl.ds(r0, C)], db, sem)

    def wait_in(xb, db, sem):
        pltpu.make_async_copy(x_hbm.at[pl.ds(0, C)], xb, sem).wait()
        pltpu.make_async_copy(d_hbm.at[pl.ds(0, C)], db, sem).wait()

    def start_out(blk, ob, sem):
        r0 = row_base + blk * C
        pltpu.async_copy(ob, out_hbm.at[pl.ds(r0, C)], sem)

    def wait_out(ob, sem):
        pltpu.make_async_copy(ob, out_hbm.at[pl.ds(0, C)], sem).wait()

    def gather(xb, db, ob):
        for r in range(C):
            xrow = xb.at[r]
            drow = db.at[r]
            orow = ob.at[r]

            @plsc.parallel_loop(0, CHUNKS, unroll=UNROLL)
            def _chunk(jj, xrow=xrow, drow=drow, orow=orow):
                idx = drow[pl.ds(jj * L, L)]
                orow[pl.ds(jj * L, L)] = plsc.load_gather(xrow, [idx])

    start_in(0, x0, d0, si0)

    def sb_body(sb, _):
        b0 = 2 * sb

        start_in(b0 + 1, x1, d1, si1)
        wait_in(x0, d0, si0)

        @pl.when(sb > 0)
        def _():
            wait_out(o0, so0)

        gather(x0, d0, o0)
        start_out(b0, o0, so0)

        @pl.when(sb < NSB - 1)
        def _():
            start_in(b0 + 2, x0, d0, si0)

        wait_in(x1, d1, si1)

        @pl.when(sb > 0)
        def _():
            wait_out(o1, so1)

        gather(x1, d1, o1)
        start_out(b0 + 1, o1, so1)
        return 0

    lax.fori_loop(0, NSB, sb_body, 0)
    wait_out(o0, so0)
    wait_out(o1, so1)


@jax.jit
def kernel(x, d):
    d32 = d.astype(jnp.int32)
    run = pl.kernel(
        _sc_gather_body,
        out_type=jax.ShapeDtypeStruct((N_ROWS, N_IDX), jnp.float32),
        mesh=plsc.VectorSubcoreMesh(core_axis_name="c", subcore_axis_name="s"),
        compiler_params=pltpu.CompilerParams(
            needs_layout_passes=False, use_tc_tiling_on_sc=False),
        scratch_types=[
            pltpu.VMEM((C, N_COLS), jnp.float32),
            pltpu.VMEM((C, N_COLS), jnp.float32),
            pltpu.VMEM((C, N_IDX), jnp.int32),
            pltpu.VMEM((C, N_IDX), jnp.int32),
            pltpu.VMEM((C, N_IDX), jnp.float32),
            pltpu.VMEM((C, N_IDX), jnp.float32),
            pltpu.SemaphoreType.DMA,
            pltpu.SemaphoreType.DMA,
            pltpu.SemaphoreType.DMA,
            pltpu.SemaphoreType.DMA,
        ],
    )
    return run(x, d32)


# speedup vs baseline: 3.0969x; 3.0969x over previous
import jax
import jax.numpy as jnp
from jax import lax
from jax.experimental import pallas as pl
from jax.experimental.pallas import tpu as pltpu
from jax.experimental.pallas import tpu_sc as plsc

N_ROWS = 16384
N_COLS = 4096
N_IDX = 2048

NC = 2
NS = 16
NW = NC * NS
L = 16

ROWS_PER_W = N_ROWS // NW
C = 8
NBLK = ROWS_PER_W // C
NSB = NBLK // 2
CHUNKS = N_IDX // L
UNROLL = 8


def _sc_gather_body(x_hbm, d_hbm, out_hbm,
                    x0, x1, d0, d1, o0, si0, si1, so0):
    wid = lax.axis_index("s") * NC + lax.axis_index("c")
    row_base = wid * ROWS_PER_W

    def start_in(blk, xb, db, sem):
        r0 = row_base + blk * C
        pltpu.async_copy(x_hbm.at[pl.ds(r0, C)], xb, sem)
        for r in range(C):
            pltpu.async_copy(d_hbm.at[r0 + r],
                             db.at[pl.ds(r * N_IDX, N_IDX)], sem)

    def wait_in(xb, db, sem):
        pltpu.make_async_copy(x_hbm.at[pl.ds(0, C)], xb, sem).wait()
        for r in range(C):
            pltpu.make_async_copy(
                d_hbm.at[0], db.at[pl.ds(r * N_IDX, N_IDX)], sem).wait()

    def start_out(blk, ob, sem):
        r0 = row_base + blk * C
        for r in range(C):
            pltpu.async_copy(ob.at[pl.ds(r * N_IDX, N_IDX)],
                             out_hbm.at[r0 + r], sem)

    def wait_out(ob, sem):
        for r in range(C):
            pltpu.make_async_copy(
                ob.at[pl.ds(r * N_IDX, N_IDX)], out_hbm.at[0], sem).wait()

    def gather(xb, db, ob):
        for r in range(C):
            row_ids = jnp.full((L,), r, jnp.int32)

            @plsc.parallel_loop(0, CHUNKS, unroll=UNROLL)
            def _chunk(jj, r=r, row_ids=row_ids, xb=xb, db=db, ob=ob):
                off = r * N_IDX + jj * L
                idx = db[pl.ds(off, L)]
                ob[pl.ds(off, L)] = plsc.load_gather(xb, [row_ids, idx])

    start_in(0, x0, d0, si0)

    def sb_body(sb, _):
        b0 = 2 * sb

        start_in(b0 + 1, x1, d1, si1)
        wait_in(x0, d0, si0)

        @pl.when(sb > 0)
        def _():
            wait_out(o0, so0)

        gather(x0, d0, o0)
        start_out(b0, o0, so0)

        @pl.when(sb < NSB - 1)
        def _():
            start_in(b0 + 2, x0, d0, si0)

        wait_in(x1, d1, si1)
        wait_out(o0, so0)
        gather(x1, d1, o0)
        start_out(b0 + 1, o0, so0)
        return 0

    lax.fori_loop(0, NSB, sb_body, 0)
    wait_out(o0, so0)


@jax.jit
def kernel(x, d):
    d32 = d.astype(jnp.int32)
    run = pl.kernel(
        _sc_gather_body,
        out_type=jax.ShapeDtypeStruct((N_ROWS, N_IDX), jnp.float32),
        mesh=plsc.VectorSubcoreMesh(core_axis_name="c", subcore_axis_name="s"),
        compiler_params=pltpu.CompilerParams(needs_layout_passes=False),
        scratch_types=[
            pltpu.VMEM((C, N_COLS), jnp.float32),
            pltpu.VMEM((C, N_COLS), jnp.float32),
            pltpu.VMEM((C * N_IDX,), jnp.int32),
            pltpu.VMEM((C * N_IDX,), jnp.int32),
            pltpu.VMEM((C * N_IDX,), jnp.float32),
            pltpu.SemaphoreType.DMA,
            pltpu.SemaphoreType.DMA,
            pltpu.SemaphoreType.DMA,
        ],
    )
    return run(x, d32)
